# baseline (speedup 1.0000x reference)
import functools

import jax
import jax.numpy as jnp
from jax import lax
from jax.experimental import pallas as pl
from jax.experimental.pallas import tpu as pltpu
from jax.experimental.pallas import tpu_sc as plsc

_NC = 2
_NS = 16
_NW = _NC * _NS

_PARTS = 5


def _node_tables_body(x_ref, ws_ref, bs_ref, wd_ref, bd_ref, wu_ref, wv_ref,
                      a_ref, b_ref):
    xb = x_ref[...]
    sf = jnp.dot(xb, ws_ref[...], preferred_element_type=jnp.float32) + bs_ref[...]
    df = jnp.dot(xb, wd_ref[...], preferred_element_type=jnp.float32) + bd_ref[...]
    a_ref[...] = jnp.dot(sf, wu_ref[...], preferred_element_type=jnp.float32)
    b_ref[...] = jnp.dot(df, wv_ref[...], preferred_element_type=jnp.float32)


def _node_tables(x, W_src, b_src, W_dst, b_dst, W1u, W1v, blk=1000):
    n, d_in = x.shape
    h = W_src.shape[1]
    rep = lambda i: (0, 0)
    return pl.pallas_call(
        _node_tables_body,
        grid=(n // blk,),
        in_specs=[
            pl.BlockSpec((blk, d_in), lambda i: (i, 0)),
            pl.BlockSpec((d_in, h), rep),
            pl.BlockSpec((1, h), rep),
            pl.BlockSpec((d_in, h), rep),
            pl.BlockSpec((1, h), rep),
            pl.BlockSpec((h, h), rep),
            pl.BlockSpec((h, h), rep),
        ],
        out_specs=[pl.BlockSpec((blk, h), lambda i: (i, 0))] * 2,
        out_shape=[jax.ShapeDtypeStruct((n, h), jnp.float32)] * 2,
    )(x, W_src, b_src.reshape(1, h), W_dst, b_dst.reshape(1, h), W1u, W1v)


def _make_gather(n, e, h, chunk=80, nbuf=5):
    per_w = e // _NW
    n_chunks = per_w // chunk
    n_groups = n_chunks // nbuf
    assert n_chunks % nbuf == 0 and per_w % chunk == 0 and chunk % 8 == 0
    mesh = plsc.VectorSubcoreMesh(core_axis_name="c", subcore_axis_name="s")

    scratch = [pltpu.VMEM((nbuf, chunk), jnp.int32),
               pltpu.VMEM((nbuf, chunk), jnp.int32)]
    scratch += [pltpu.VMEM((chunk, h), jnp.float32)] * (2 * nbuf)
    scratch += [pltpu.SemaphoreType.DMA] * (4 * nbuf)

    @functools.partial(
        pl.kernel,
        out_type=jax.ShapeDtypeStruct((e, h), jnp.float32),
        mesh=mesh,
        scratch_types=scratch,
    )
    def gather_k(a_hbm, b_hbm, src_hbm, dst_hbm, g_hbm, si, di, *rest):
        ra = rest[:nbuf]
        rb = rest[nbuf:2 * nbuf]
        semi = rest[2 * nbuf:3 * nbuf]
        sema = rest[3 * nbuf:4 * nbuf]
        semb = rest[4 * nbuf:5 * nbuf]
        semw = rest[5 * nbuf:6 * nbuf]
        wid = lax.axis_index("s") * _NC + lax.axis_index("c")
        base = wid * per_w

        def idx_start(i, j):
            off = base + i * chunk
            pltpu.async_copy(src_hbm.at[pl.ds(off, chunk)], si.at[j], semi[j])
            pltpu.async_copy(dst_hbm.at[pl.ds(off, chunk)], di.at[j], semi[j])

        def idx_wait(j):
            pltpu.make_async_copy(src_hbm.at[pl.ds(base, chunk)], si.at[j],
                                  semi[j]).wait()
            pltpu.make_async_copy(dst_hbm.at[pl.ds(base, chunk)], di.at[j],
                                  semi[j]).wait()

        def w_wait(j):
            pltpu.make_async_copy(ra[j], g_hbm.at[pl.ds(base, chunk)],
                                  semw[j]).wait()

        def retire(i, j):
            pltpu.make_async_copy(a_hbm.at[si.at[j]], ra[j], sema[j]).wait()
            pltpu.make_async_copy(b_hbm.at[di.at[j]], rb[j], semb[j]).wait()

            def addrow(r, carry):
                for q in range(h // 16):
                    sl = pl.ds(q * 16, 16)
                    ra[j][r, sl] = ra[j][r, sl] + rb[j][r, sl]
                return carry

            lax.fori_loop(0, chunk, addrow, 0)
            off = base + i * chunk
            pltpu.async_copy(ra[j], g_hbm.at[pl.ds(off, chunk)], semw[j])

        for j in range(nbuf):
            idx_start(j, j)

        def body(g, carry):
            for j in range(nbuf):
                i = g * nbuf + j
                jp = (j - 1) % nbuf

                @pl.when(g > 0)
                def _free_slot():
                    w_wait(j)

                idx_wait(j)
                pltpu.async_copy(a_hbm.at[si.at[j]], ra[j], sema[j])
                pltpu.async_copy(b_hbm.at[di.at[j]], rb[j], semb[j])

                @pl.when(i > 0)
                def _retire_prev():
                    retire(i - 1, jp)

                    @pl.when(i - 1 + nbuf < n_chunks)
                    def _refill_idx():
                        idx_start(i - 1 + nbuf, jp)

            return carry

        lax.fori_loop(0, n_groups, body, 0)
        retire(n_chunks - 1, (n_chunks - 1) % nbuf)
        for j in range(nbuf):
            w_wait(j)

    return gather_k


def _edge_mlp_body(g_in_ref, ea_ref, w1e_ref, b1_ref, w2_ref, b2_ref,
                   w3_ref, b3_ref, g_ref, bb_ref, h_ref):
    pre = (g_in_ref[...]
           + jnp.dot(ea_ref[...], w1e_ref[...], preferred_element_type=jnp.float32)
           + b1_ref[...])
    h1 = jax.nn.gelu(pre)
    h2 = jax.nn.gelu(jnp.dot(h1, w2_ref[...], preferred_element_type=jnp.float32)
                     + b2_ref[...])
    h3 = jnp.dot(h2, w3_ref[...], preferred_element_type=jnp.float32) + b3_ref[...]
    mu = jnp.mean(h3, axis=-1, keepdims=True)
    var = jnp.mean((h3 - mu) ** 2, axis=-1, keepdims=True)
    h_ref[...] = (h3 - mu) * lax.rsqrt(var + 1e-5) * g_ref[...] + bb_ref[...]


def _edge_mlp_part(h_carry, Gp, ea_p, W1e, b_e1, W_e2, b_e2, W_e3, b_e3,
                   ln_g, ln_b, part, e_total, blk=2000):
    ep, h = Gp.shape
    d_edge = ea_p.shape[1]
    nblk = ep // blk
    off = part * nblk
    rep = lambda i: (0, 0)
    body = _edge_mlp_body
    in_specs = [
        pl.BlockSpec((blk, h), lambda i: (i, 0)),
        pl.BlockSpec((blk, d_edge), lambda i: (i, 0)),
        pl.BlockSpec((d_edge, h), rep),
        pl.BlockSpec((1, h), rep),
        pl.BlockSpec((h, h), rep),
        pl.BlockSpec((1, h), rep),
        pl.BlockSpec((h, h), rep),
        pl.BlockSpec((1, h), rep),
        pl.BlockSpec((1, h), rep),
        pl.BlockSpec((1, h), rep),
    ]
    args = [Gp, ea_p, W1e, b_e1.reshape(1, h), W_e2, b_e2.reshape(1, h),
            W_e3, b_e3.reshape(1, h), ln_g.reshape(1, h), ln_b.reshape(1, h)]
    kwargs = {}
    if h_carry is not None:
        body = lambda hc_ref, *refs: _edge_mlp_body(*refs)
        in_specs = [pl.BlockSpec(memory_space=pl.ANY)] + in_specs
        args = [h_carry] + args
        kwargs = dict(input_output_aliases={0: 0})
    return pl.pallas_call(
        body,
        grid=(nblk,),
        in_specs=in_specs,
        out_specs=pl.BlockSpec((blk, h), lambda i: (off + i, 0)),
        out_shape=jax.ShapeDtypeStruct((e_total, h), jnp.float32),
        **kwargs,
    )(*args)


def _make_scatter(n, e, h, chunk=40, nbuf=5):
    per_w = e // _NW
    n_chunks = per_w // chunk
    n_groups = n_chunks // nbuf
    assert n_chunks % nbuf == 0 and per_w % chunk == 0 and chunk % 8 == 0
    rpt = (n // _NS) // 8 * 8
    tail = n - rpt * _NS
    mesh = plsc.VectorSubcoreMesh(core_axis_name="c", subcore_axis_name="s")

    scratch = [pltpu.VMEM((nbuf, chunk), jnp.int32)]
    scratch += [pltpu.VMEM((chunk, h), jnp.float32)] * nbuf
    scratch += [pltpu.VMEM_SHARED((n, h), jnp.float32)]
    scratch += [pltpu.SemaphoreType.DMA] * (2 * nbuf)

    @functools.partial(
        pl.kernel,
        out_type=jax.ShapeDtypeStruct((_NC, n, h), jnp.float32),
        mesh=mesh,
        scratch_types=scratch,
    )
    def scatter_k(h_hbm, dst_hbm, zero_hbm, out_hbm, di, *rest):
        hb = rest[:nbuf]
        acc = rest[nbuf]
        seml = rest[nbuf + 1:2 * nbuf + 1]
        sems = rest[2 * nbuf + 1:3 * nbuf + 1]
        c = lax.axis_index("c")
        s = lax.axis_index("s")
        wid = s * _NC + c
        base = wid * per_w
        r0 = s * rpt

        def load_start(i, j):
            off = base + i * chunk
            pltpu.async_copy(dst_hbm.at[pl.ds(off, chunk)], di.at[j], seml[j])
            pltpu.async_copy(h_hbm.at[pl.ds(off, chunk)], hb[j], seml[j])

        def load_wait(j):
            pltpu.make_async_copy(dst_hbm.at[pl.ds(base, chunk)], di.at[j],
                                  seml[j]).wait()
            pltpu.make_async_copy(h_hbm.at[pl.ds(base, chunk)], hb[j],
                                  seml[j]).wait()

        def scat_wait(j):
            pltpu.make_async_copy(hb[j], acc.at[di.at[j]], sems[j]).wait()

        pltpu.sync_copy(zero_hbm.at[pl.ds(r0, rpt)], acc.at[pl.ds(r0, rpt)])
        if tail:
            @pl.when(s == 0)
            def _zero_tail():
                pltpu.sync_copy(zero_hbm.at[pl.ds(n - tail, tail)],
                                acc.at[pl.ds(n - tail, tail)])
        for j in range(nbuf):
            load_start(j, j)
        plsc.subcore_barrier()

        def body(g, carry):
            for j in range(nbuf):
                i = g * nbuf + j
                jp = (j - 1) % nbuf
                load_wait(j)
                pltpu.async_copy(hb[j], acc.at[di.at[j]], sems[j], add=True)

                @pl.when(i > 0)
                def _retire_prev():
                    scat_wait(jp)

                    @pl.when(i - 1 + nbuf < n_chunks)
                    def _refill():
                        load_start(i - 1 + nbuf, jp)

            return carry

        lax.fori_loop(0, n_groups, body, 0)
        scat_wait((n_chunks - 1) % nbuf)
        plsc.subcore_barrier()
        pltpu.sync_copy(acc.at[pl.ds(r0, rpt)], out_hbm.at[c, pl.ds(r0, rpt)])
        if tail:
            @pl.when(s == 0)
            def _write_tail():
                pltpu.sync_copy(acc.at[pl.ds(n - tail, tail)],
                                out_hbm.at[c, pl.ds(n - tail, tail)])

    return scatter_k


def _out_body(p_ref, wo_ref, bo_ref, o_ref):
    s = p_ref[0] + p_ref[1]
    o_ref[...] = jnp.dot(s, wo_ref[...], preferred_element_type=jnp.float32) + bo_ref[...]


def _out_proj(partials, W_out, b_out, blk=1000):
    _, n, h = partials.shape
    d_out = W_out.shape[1]
    return pl.pallas_call(
        _out_body,
        grid=(n // blk,),
        in_specs=[
            pl.BlockSpec((_NC, blk, h), lambda i: (0, i, 0)),
            pl.BlockSpec((h, d_out), lambda i: (0, 0)),
            pl.BlockSpec((1, d_out), lambda i: (0, 0)),
        ],
        out_specs=pl.BlockSpec((blk, d_out), lambda i: (i, 0)),
        out_shape=jax.ShapeDtypeStruct((n, d_out), jnp.float32),
    )(partials, W_out, b_out.reshape(1, d_out))


def kernel(x, edge_attr, W_src, b_src, W_dst, b_dst, W_e1, b_e1, W_e2, b_e2,
           W_e3, b_e3, ln_g, ln_b, W_out, b_out, edge_index):
    n, _ = x.shape
    e, _ = edge_attr.shape
    h = W_src.shape[1]
    src = edge_index[0]
    dst = edge_index[1]
    W1u = W_e1[:h]
    W1v = W_e1[h:2 * h]
    W1e = W_e1[2 * h:]

    A, B = _node_tables(x, W_src, b_src, W_dst, b_dst, W1u, W1v)

    ep = e // _PARTS
    gather = _make_gather(n, ep, h)
    Gs = [gather(A, B, src[p * ep:(p + 1) * ep], dst[p * ep:(p + 1) * ep])
          for p in range(_PARTS)]
    hmsg = None
    for p in range(_PARTS):
        sl = slice(p * ep, (p + 1) * ep)
        hmsg = _edge_mlp_part(hmsg, Gs[p], edge_attr[sl], W1e, b_e1, W_e2, b_e2,
                              W_e3, b_e3, ln_g, ln_b, p, e)

    zeros = jnp.zeros((n, h), jnp.float32)
    partials = _make_scatter(n, e, h)(hmsg, dst, zeros)
    return _out_proj(partials, W_out, b_out)


# speedup vs baseline: 1.0214x; 1.0214x over previous
import functools

import jax
import jax.numpy as jnp
from jax import lax
from jax.experimental import pallas as pl
from jax.experimental.pallas import tpu as pltpu
from jax.experimental.pallas import tpu_sc as plsc

_NC = 2
_NS = 16
_NW = _NC * _NS

_PARTS = 1


def _node_tables_body(x_ref, ws_ref, bs_ref, wd_ref, bd_ref, wu_ref, wv_ref,
                      a_ref, b_ref):
    xb = x_ref[...]
    sf = jnp.dot(xb, ws_ref[...], preferred_element_type=jnp.float32) + bs_ref[...]
    df = jnp.dot(xb, wd_ref[...], preferred_element_type=jnp.float32) + bd_ref[...]
    a_ref[...] = jnp.dot(sf, wu_ref[...], preferred_element_type=jnp.float32)
    b_ref[...] = jnp.dot(df, wv_ref[...], preferred_element_type=jnp.float32)


def _node_tables(x, W_src, b_src, W_dst, b_dst, W1u, W1v, blk=1000):
    n, d_in = x.shape
    h = W_src.shape[1]
    rep = lambda i: (0, 0)
    return pl.pallas_call(
        _node_tables_body,
        grid=(n // blk,),
        in_specs=[
            pl.BlockSpec((blk, d_in), lambda i: (i, 0)),
            pl.BlockSpec((d_in, h), rep),
            pl.BlockSpec((1, h), rep),
            pl.BlockSpec((d_in, h), rep),
            pl.BlockSpec((1, h), rep),
            pl.BlockSpec((h, h), rep),
            pl.BlockSpec((h, h), rep),
        ],
        out_specs=[pl.BlockSpec((blk, h), lambda i: (i, 0))] * 2,
        out_shape=[jax.ShapeDtypeStruct((n, h), jnp.float32)] * 2,
    )(x, W_src, b_src.reshape(1, h), W_dst, b_dst.reshape(1, h), W1u, W1v)


def _make_gather(n, e, h, chunk=80, nbuf=5):
    per_w = e // _NW
    n_chunks = per_w // chunk
    n_groups = n_chunks // nbuf
    assert n_chunks % nbuf == 0 and per_w % chunk == 0 and chunk % 8 == 0
    mesh = plsc.VectorSubcoreMesh(core_axis_name="c", subcore_axis_name="s")

    scratch = [pltpu.VMEM((nbuf, chunk), jnp.int32),
               pltpu.VMEM((nbuf, chunk), jnp.int32)]
    scratch += [pltpu.VMEM((chunk, h), jnp.float32)] * (2 * nbuf)
    scratch += [pltpu.SemaphoreType.DMA] * (4 * nbuf)

    @functools.partial(
        pl.kernel,
        out_type=jax.ShapeDtypeStruct((e, h), jnp.float32),
        mesh=mesh,
        scratch_types=scratch,
    )
    def gather_k(a_hbm, b_hbm, src_hbm, dst_hbm, g_hbm, si, di, *rest):
        ra = rest[:nbuf]
        rb = rest[nbuf:2 * nbuf]
        semi = rest[2 * nbuf:3 * nbuf]
        sema = rest[3 * nbuf:4 * nbuf]
        semb = rest[4 * nbuf:5 * nbuf]
        semw = rest[5 * nbuf:6 * nbuf]
        wid = lax.axis_index("s") * _NC + lax.axis_index("c")
        base = wid * per_w

        def idx_start(i, j):
            off = base + i * chunk
            pltpu.async_copy(src_hbm.at[pl.ds(off, chunk)], si.at[j], semi[j])
            pltpu.async_copy(dst_hbm.at[pl.ds(off, chunk)], di.at[j], semi[j])

        def idx_wait(j):
            pltpu.make_async_copy(src_hbm.at[pl.ds(base, chunk)], si.at[j],
                                  semi[j]).wait()
            pltpu.make_async_copy(dst_hbm.at[pl.ds(base, chunk)], di.at[j],
                                  semi[j]).wait()

        def w_wait(j):
            pltpu.make_async_copy(ra[j], g_hbm.at[pl.ds(base, chunk)],
                                  semw[j]).wait()

        def retire(i, j):
            pltpu.make_async_copy(a_hbm.at[si.at[j]], ra[j], sema[j]).wait()
            pltpu.make_async_copy(b_hbm.at[di.at[j]], rb[j], semb[j]).wait()

            def addrow(r, carry):
                for q in range(h // 16):
                    sl = pl.ds(q * 16, 16)
                    ra[j][r, sl] = ra[j][r, sl] + rb[j][r, sl]
                return carry

            lax.fori_loop(0, chunk, addrow, 0)
            off = base + i * chunk
            pltpu.async_copy(ra[j], g_hbm.at[pl.ds(off, chunk)], semw[j])

        for j in range(nbuf):
            idx_start(j, j)

        def body(g, carry):
            for j in range(nbuf):
                i = g * nbuf + j
                jp = (j - 1) % nbuf

                @pl.when(g > 0)
                def _free_slot():
                    w_wait(j)

                idx_wait(j)
                pltpu.async_copy(a_hbm.at[si.at[j]], ra[j], sema[j])
                pltpu.async_copy(b_hbm.at[di.at[j]], rb[j], semb[j])

                @pl.when(i > 0)
                def _retire_prev():
                    retire(i - 1, jp)

                    @pl.when(i - 1 + nbuf < n_chunks)
                    def _refill_idx():
                        idx_start(i - 1 + nbuf, jp)

            return carry

        lax.fori_loop(0, n_groups, body, 0)
        retire(n_chunks - 1, (n_chunks - 1) % nbuf)
        for j in range(nbuf):
            w_wait(j)

    return gather_k


def _edge_mlp_body(g_in_ref, ea_ref, w1e_ref, b1_ref, w2_ref, b2_ref,
                   w3_ref, b3_ref, g_ref, bb_ref, h_ref):
    blk, h = h_ref.shape
    ea_term = jnp.dot(ea_ref[...], w1e_ref[...],
                      preferred_element_type=jnp.float32).reshape(blk, h)
    pre = g_in_ref[...] + ea_term + b1_ref[...]
    h1 = jax.nn.gelu(pre)
    h2 = jax.nn.gelu(jnp.dot(h1, w2_ref[...], preferred_element_type=jnp.float32)
                     + b2_ref[...])
    h3 = jnp.dot(h2, w3_ref[...], preferred_element_type=jnp.float32) + b3_ref[...]
    mu = jnp.mean(h3, axis=-1, keepdims=True)
    var = jnp.mean((h3 - mu) ** 2, axis=-1, keepdims=True)
    h_ref[...] = (h3 - mu) * lax.rsqrt(var + 1e-5) * g_ref[...] + bb_ref[...]


def _edge_mlp_part(h_carry, Gp, ea8, W1e_big, b_e1, W_e2, b_e2, W_e3, b_e3,
                   ln_g, ln_b, part, e_total, blk=3200):
    ep, h = Gp.shape
    d8 = ea8.shape[1]
    nblk = ep // blk
    off = part * nblk
    rep = lambda i: (0, 0)
    body = _edge_mlp_body
    in_specs = [
        pl.BlockSpec((blk, h), lambda i: (i, 0)),
        pl.BlockSpec((blk // 8, d8), lambda i: (off + i, 0)),
        pl.BlockSpec((d8, 8 * h), rep),
        pl.BlockSpec((1, h), rep),
        pl.BlockSpec((h, h), rep),
        pl.BlockSpec((1, h), rep),
        pl.BlockSpec((h, h), rep),
        pl.BlockSpec((1, h), rep),
        pl.BlockSpec((1, h), rep),
        pl.BlockSpec((1, h), rep),
    ]
    args = [Gp, ea8, W1e_big, b_e1.reshape(1, h), W_e2, b_e2.reshape(1, h),
            W_e3, b_e3.reshape(1, h), ln_g.reshape(1, h), ln_b.reshape(1, h)]
    kwargs = {}
    if h_carry is not None:
        body = lambda hc_ref, *refs: _edge_mlp_body(*refs)
        in_specs = [pl.BlockSpec(memory_space=pl.ANY)] + in_specs
        args = [h_carry] + args
        kwargs = dict(input_output_aliases={0: 0})
    return pl.pallas_call(
        body,
        grid=(nblk,),
        in_specs=in_specs,
        out_specs=pl.BlockSpec((blk, h), lambda i: (off + i, 0)),
        out_shape=jax.ShapeDtypeStruct((e_total, h), jnp.float32),
        **kwargs,
    )(*args)


def _make_scatter(n, e, h, chunk=40, nbuf=5):
    per_w = e // _NW
    n_chunks = per_w // chunk
    n_groups = n_chunks // nbuf
    assert n_chunks % nbuf == 0 and per_w % chunk == 0 and chunk % 8 == 0
    rpt = (n // _NS) // 8 * 8
    tail = n - rpt * _NS
    mesh = plsc.VectorSubcoreMesh(core_axis_name="c", subcore_axis_name="s")

    scratch = [pltpu.VMEM((nbuf, chunk), jnp.int32)]
    scratch += [pltpu.VMEM((chunk, h), jnp.float32)] * nbuf
    scratch += [pltpu.VMEM_SHARED((n, h), jnp.float32)]
    scratch += [pltpu.SemaphoreType.DMA] * (2 * nbuf)

    @functools.partial(
        pl.kernel,
        out_type=jax.ShapeDtypeStruct((_NC, n, h), jnp.float32),
        mesh=mesh,
        scratch_types=scratch,
    )
    def scatter_k(h_hbm, dst_hbm, zero_hbm, out_hbm, di, *rest):
        hb = rest[:nbuf]
        acc = rest[nbuf]
        seml = rest[nbuf + 1:2 * nbuf + 1]
        sems = rest[2 * nbuf + 1:3 * nbuf + 1]
        c = lax.axis_index("c")
        s = lax.axis_index("s")
        wid = s * _NC + c
        base = wid * per_w
        r0 = s * rpt

        def load_start(i, j):
            off = base + i * chunk
            pltpu.async_copy(dst_hbm.at[pl.ds(off, chunk)], di.at[j], seml[j])
            pltpu.async_copy(h_hbm.at[pl.ds(off, chunk)], hb[j], seml[j])

        def load_wait(j):
            pltpu.make_async_copy(dst_hbm.at[pl.ds(base, chunk)], di.at[j],
                                  seml[j]).wait()
            pltpu.make_async_copy(h_hbm.at[pl.ds(base, chunk)], hb[j],
                                  seml[j]).wait()

        def scat_wait(j):
            pltpu.make_async_copy(hb[j], acc.at[di.at[j]], sems[j]).wait()

        pltpu.sync_copy(zero_hbm.at[pl.ds(r0, rpt)], acc.at[pl.ds(r0, rpt)])
        if tail:
            @pl.when(s == 0)
            def _zero_tail():
                pltpu.sync_copy(zero_hbm.at[pl.ds(n - tail, tail)],
                                acc.at[pl.ds(n - tail, tail)])
        for j in range(nbuf):
            load_start(j, j)
        plsc.subcore_barrier()

        def body(g, carry):
            for j in range(nbuf):
                i = g * nbuf + j
                jp = (j - 1) % nbuf
                load_wait(j)
                pltpu.async_copy(hb[j], acc.at[di.at[j]], sems[j], add=True)

                @pl.when(i > 0)
                def _retire_prev():
                    scat_wait(jp)

                    @pl.when(i - 1 + nbuf < n_chunks)
                    def _refill():
                        load_start(i - 1 + nbuf, jp)

            return carry

        lax.fori_loop(0, n_groups, body, 0)
        scat_wait((n_chunks - 1) % nbuf)
        plsc.subcore_barrier()
        pltpu.sync_copy(acc.at[pl.ds(r0, rpt)], out_hbm.at[c, pl.ds(r0, rpt)])
        if tail:
            @pl.when(s == 0)
            def _write_tail():
                pltpu.sync_copy(acc.at[pl.ds(n - tail, tail)],
                                out_hbm.at[c, pl.ds(n - tail, tail)])

    return scatter_k


def _out_body(p_ref, wo_ref, bo_ref, o_ref):
    s = p_ref[0] + p_ref[1]
    o_ref[...] = jnp.dot(s, wo_ref[...], preferred_element_type=jnp.float32) + bo_ref[...]


def _out_proj(partials, W_out, b_out, blk=1000):
    _, n, h = partials.shape
    d_out = W_out.shape[1]
    return pl.pallas_call(
        _out_body,
        grid=(n // blk,),
        in_specs=[
            pl.BlockSpec((_NC, blk, h), lambda i: (0, i, 0)),
            pl.BlockSpec((h, d_out), lambda i: (0, 0)),
            pl.BlockSpec((1, d_out), lambda i: (0, 0)),
        ],
        out_specs=pl.BlockSpec((blk, d_out), lambda i: (i, 0)),
        out_shape=jax.ShapeDtypeStruct((n, d_out), jnp.float32),
    )(partials, W_out, b_out.reshape(1, d_out))


def kernel(x, edge_attr, W_src, b_src, W_dst, b_dst, W_e1, b_e1, W_e2, b_e2,
           W_e3, b_e3, ln_g, ln_b, W_out, b_out, edge_index):
    n, _ = x.shape
    e, _ = edge_attr.shape
    h = W_src.shape[1]
    src = edge_index[0]
    dst = edge_index[1]
    W1u = W_e1[:h]
    W1v = W_e1[h:2 * h]
    W1e = W_e1[2 * h:]

    A, B = _node_tables(x, W_src, b_src, W_dst, b_dst, W1u, W1v)

    d_edge = edge_attr.shape[1]
    ea8 = edge_attr.reshape(e // 8, 8 * d_edge)
    W1e_big = jnp.zeros((8 * d_edge, 8 * h), jnp.float32)
    for k in range(8):
        W1e_big = W1e_big.at[k * d_edge:(k + 1) * d_edge,
                             k * h:(k + 1) * h].set(W1e)

    ep = e // _PARTS
    gather = _make_gather(n, ep, h)
    Gs = [gather(A, B, src[p * ep:(p + 1) * ep], dst[p * ep:(p + 1) * ep])
          for p in range(_PARTS)]
    hmsg = None
    for p in range(_PARTS):
        hmsg = _edge_mlp_part(hmsg, Gs[p], ea8, W1e_big, b_e1, W_e2, b_e2,
                              W_e3, b_e3, ln_g, ln_b, p, e)

    zeros = jnp.zeros((n, h), jnp.float32)
    partials = _make_scatter(n, e, h)(hmsg, dst, zeros)
    return _out_proj(partials, W_out, b_out)


# speedup vs baseline: 1.0411x; 1.0193x over previous
import functools

import jax
import jax.numpy as jnp
from jax import lax
from jax.experimental import pallas as pl
from jax.experimental.pallas import tpu as pltpu
from jax.experimental.pallas import tpu_sc as plsc

_NC = 2
_NS = 16
_NW = _NC * _NS

_PARTS = 1


def _node_tables_body(x_ref, ws_ref, bs_ref, wd_ref, bd_ref, wu_ref, wv_ref,
                      a_ref, b_ref):
    xb = x_ref[...]
    sf = jnp.dot(xb, ws_ref[...], preferred_element_type=jnp.float32) + bs_ref[...]
    df = jnp.dot(xb, wd_ref[...], preferred_element_type=jnp.float32) + bd_ref[...]
    a_ref[...] = jnp.dot(sf, wu_ref[...], preferred_element_type=jnp.float32)
    b_ref[...] = jnp.dot(df, wv_ref[...], preferred_element_type=jnp.float32)


def _node_tables(x, W_src, b_src, W_dst, b_dst, W1u, W1v, blk=1000):
    n, d_in = x.shape
    h = W_src.shape[1]
    rep = lambda i: (0, 0)
    return pl.pallas_call(
        _node_tables_body,
        grid=(n // blk,),
        in_specs=[
            pl.BlockSpec((blk, d_in), lambda i: (i, 0)),
            pl.BlockSpec((d_in, h), rep),
            pl.BlockSpec((1, h), rep),
            pl.BlockSpec((d_in, h), rep),
            pl.BlockSpec((1, h), rep),
            pl.BlockSpec((h, h), rep),
            pl.BlockSpec((h, h), rep),
        ],
        out_specs=[pl.BlockSpec((blk, h), lambda i: (i, 0))] * 2,
        out_shape=[jax.ShapeDtypeStruct((n, h), jnp.float32)] * 2,
    )(x, W_src, b_src.reshape(1, h), W_dst, b_dst.reshape(1, h), W1u, W1v)


def _make_gather(n, e, h, chunk=80, nbuf=5):
    per_w = e // _NW
    n_chunks = per_w // chunk
    n_groups = n_chunks // nbuf
    assert n_chunks % nbuf == 0 and per_w % chunk == 0 and chunk % 8 == 0
    mesh = plsc.VectorSubcoreMesh(core_axis_name="c", subcore_axis_name="s")

    scratch = [pltpu.VMEM((nbuf, chunk), jnp.int32),
               pltpu.VMEM((nbuf, chunk), jnp.int32)]
    scratch += [pltpu.VMEM((chunk, h), jnp.float32)] * (2 * nbuf)
    scratch += [pltpu.SemaphoreType.DMA] * (4 * nbuf)

    @functools.partial(
        pl.kernel,
        out_type=jax.ShapeDtypeStruct((e, h), jnp.float32),
        mesh=mesh,
        scratch_types=scratch,
    )
    def gather_k(a_hbm, b_hbm, src_hbm, dst_hbm, g_hbm, si, di, *rest):
        ra = rest[:nbuf]
        rb = rest[nbuf:2 * nbuf]
        semi = rest[2 * nbuf:3 * nbuf]
        sema = rest[3 * nbuf:4 * nbuf]
        semb = rest[4 * nbuf:5 * nbuf]
        semw = rest[5 * nbuf:6 * nbuf]
        wid = lax.axis_index("s") * _NC + lax.axis_index("c")
        base = wid * per_w

        def idx_start(i, j):
            off = base + i * chunk
            pltpu.async_copy(src_hbm.at[pl.ds(off, chunk)], si.at[j], semi[j])
            pltpu.async_copy(dst_hbm.at[pl.ds(off, chunk)], di.at[j], semi[j])

        def idx_wait(j):
            pltpu.make_async_copy(src_hbm.at[pl.ds(base, chunk)], si.at[j],
                                  semi[j]).wait()
            pltpu.make_async_copy(dst_hbm.at[pl.ds(base, chunk)], di.at[j],
                                  semi[j]).wait()

        def w_wait(j):
            pltpu.make_async_copy(ra[j], g_hbm.at[pl.ds(base, chunk)],
                                  semw[j]).wait()

        def retire(i, j):
            pltpu.make_async_copy(a_hbm.at[si.at[j]], ra[j], sema[j]).wait()
            pltpu.make_async_copy(b_hbm.at[di.at[j]], rb[j], semb[j]).wait()

            def addrow(r, carry):
                for q in range(h // 16):
                    sl = pl.ds(q * 16, 16)
                    ra[j][r, sl] = ra[j][r, sl] + rb[j][r, sl]
                return carry

            lax.fori_loop(0, chunk, addrow, 0)
            off = base + i * chunk
            pltpu.async_copy(ra[j], g_hbm.at[pl.ds(off, chunk)], semw[j])

        for j in range(nbuf):
            idx_start(j, j)

        def body(g, carry):
            for j in range(nbuf):
                i = g * nbuf + j
                jp = (j - 1) % nbuf

                @pl.when(g > 0)
                def _free_slot():
                    w_wait(j)

                idx_wait(j)
                pltpu.async_copy(a_hbm.at[si.at[j]], ra[j], sema[j])
                pltpu.async_copy(b_hbm.at[di.at[j]], rb[j], semb[j])

                @pl.when(i > 0)
                def _retire_prev():
                    retire(i - 1, jp)

                    @pl.when(i - 1 + nbuf < n_chunks)
                    def _refill_idx():
                        idx_start(i - 1 + nbuf, jp)

            return carry

        lax.fori_loop(0, n_groups, body, 0)
        retire(n_chunks - 1, (n_chunks - 1) % nbuf)
        for j in range(nbuf):
            w_wait(j)

    return gather_k


def _edge_mlp_body(g_in_ref, ea_ref, w1e_ref, b1_ref, w2_ref, b2_ref,
                   w3_ref, b3_ref, g_ref, bb_ref, h_ref):
    blk, h = h_ref.shape
    ea_term = jnp.dot(ea_ref[...], w1e_ref[...],
                      preferred_element_type=jnp.float32).reshape(blk, h)
    pre = g_in_ref[...] + ea_term + b1_ref[...]
    h1 = jax.nn.gelu(pre)
    h2 = jax.nn.gelu(jnp.dot(h1, w2_ref[...], preferred_element_type=jnp.float32)
                     + b2_ref[...])
    h3 = jnp.dot(h2, w3_ref[...], preferred_element_type=jnp.float32) + b3_ref[...]
    mu = jnp.mean(h3, axis=-1, keepdims=True)
    var = jnp.mean((h3 - mu) ** 2, axis=-1, keepdims=True)
    h_ref[...] = (h3 - mu) * lax.rsqrt(var + 1e-5) * g_ref[...] + bb_ref[...]


def _edge_mlp_part(h_carry, Gp, ea8, W1e_big, b_e1, W_e2, b_e2, W_e3, b_e3,
                   ln_g, ln_b, part, e_total, blk=6400):
    ep, h = Gp.shape
    d8 = ea8.shape[1]
    nblk = ep // blk
    off = part * nblk
    rep = lambda i: (0, 0)
    body = _edge_mlp_body
    in_specs = [
        pl.BlockSpec((blk, h), lambda i: (i, 0)),
        pl.BlockSpec((blk // 8, d8), lambda i: (off + i, 0)),
        pl.BlockSpec((d8, 8 * h), rep),
        pl.BlockSpec((1, h), rep),
        pl.BlockSpec((h, h), rep),
        pl.BlockSpec((1, h), rep),
        pl.BlockSpec((h, h), rep),
        pl.BlockSpec((1, h), rep),
        pl.BlockSpec((1, h), rep),
        pl.BlockSpec((1, h), rep),
    ]
    args = [Gp, ea8, W1e_big, b_e1.reshape(1, h), W_e2, b_e2.reshape(1, h),
            W_e3, b_e3.reshape(1, h), ln_g.reshape(1, h), ln_b.reshape(1, h)]
    kwargs = {}
    if h_carry is not None:
        body = lambda hc_ref, *refs: _edge_mlp_body(*refs)
        in_specs = [pl.BlockSpec(memory_space=pl.ANY)] + in_specs
        args = [h_carry] + args
        kwargs = dict(input_output_aliases={0: 0})
    return pl.pallas_call(
        body,
        grid=(nblk,),
        in_specs=in_specs,
        out_specs=pl.BlockSpec((blk, h), lambda i: (off + i, 0)),
        out_shape=jax.ShapeDtypeStruct((e_total, h), jnp.float32),
        **kwargs,
    )(*args)


def _make_scatter(n, e, h, chunk=40, nbuf=5):
    per_w = e // _NW
    n_chunks = per_w // chunk
    n_groups = n_chunks // nbuf
    assert n_chunks % nbuf == 0 and per_w % chunk == 0 and chunk % 8 == 0
    rpt = (n // _NS) // 8 * 8
    tail = n - rpt * _NS
    mesh = plsc.VectorSubcoreMesh(core_axis_name="c", subcore_axis_name="s")

    scratch = [pltpu.VMEM((nbuf, chunk), jnp.int32)]
    scratch += [pltpu.VMEM((chunk, h), jnp.float32)] * nbuf
    scratch += [pltpu.VMEM_SHARED((n, h), jnp.float32)]
    scratch += [pltpu.SemaphoreType.DMA] * (2 * nbuf)

    @functools.partial(
        pl.kernel,
        out_type=jax.ShapeDtypeStruct((_NC, n, h), jnp.float32),
        mesh=mesh,
        scratch_types=scratch,
    )
    def scatter_k(h_hbm, dst_hbm, zero_hbm, out_hbm, di, *rest):
        hb = rest[:nbuf]
        acc = rest[nbuf]
        seml = rest[nbuf + 1:2 * nbuf + 1]
        sems = rest[2 * nbuf + 1:3 * nbuf + 1]
        c = lax.axis_index("c")
        s = lax.axis_index("s")
        wid = s * _NC + c
        base = wid * per_w
        r0 = s * rpt

        def load_start(i, j):
            off = base + i * chunk
            pltpu.async_copy(dst_hbm.at[pl.ds(off, chunk)], di.at[j], seml[j])
            pltpu.async_copy(h_hbm.at[pl.ds(off, chunk)], hb[j], seml[j])

        def load_wait(j):
            pltpu.make_async_copy(dst_hbm.at[pl.ds(base, chunk)], di.at[j],
                                  seml[j]).wait()
            pltpu.make_async_copy(h_hbm.at[pl.ds(base, chunk)], hb[j],
                                  seml[j]).wait()

        def scat_wait(j):
            pltpu.make_async_copy(hb[j], acc.at[di.at[j]], sems[j]).wait()

        pltpu.sync_copy(zero_hbm.at[pl.ds(r0, rpt)], acc.at[pl.ds(r0, rpt)])
        if tail:
            @pl.when(s == 0)
            def _zero_tail():
                pltpu.sync_copy(zero_hbm.at[pl.ds(n - tail, tail)],
                                acc.at[pl.ds(n - tail, tail)])
        for j in range(nbuf):
            load_start(j, j)
        plsc.subcore_barrier()

        def body(g, carry):
            for j in range(nbuf):
                i = g * nbuf + j
                jp = (j - 1) % nbuf
                load_wait(j)
                pltpu.async_copy(hb[j], acc.at[di.at[j]], sems[j], add=True)

                @pl.when(i > 0)
                def _retire_prev():
                    scat_wait(jp)

                    @pl.when(i - 1 + nbuf < n_chunks)
                    def _refill():
                        load_start(i - 1 + nbuf, jp)

            return carry

        lax.fori_loop(0, n_groups, body, 0)
        scat_wait((n_chunks - 1) % nbuf)
        plsc.subcore_barrier()
        pltpu.sync_copy(acc.at[pl.ds(r0, rpt)], out_hbm.at[c, pl.ds(r0, rpt)])
        if tail:
            @pl.when(s == 0)
            def _write_tail():
                pltpu.sync_copy(acc.at[pl.ds(n - tail, tail)],
                                out_hbm.at[c, pl.ds(n - tail, tail)])

    return scatter_k


def _out_body(p_ref, wo_ref, bo_ref, o_ref):
    s = p_ref[0] + p_ref[1]
    o_ref[...] = jnp.dot(s, wo_ref[...], preferred_element_type=jnp.float32) + bo_ref[...]


def _out_proj(partials, W_out, b_out, blk=1000):
    _, n, h = partials.shape
    d_out = W_out.shape[1]
    return pl.pallas_call(
        _out_body,
        grid=(n // blk,),
        in_specs=[
            pl.BlockSpec((_NC, blk, h), lambda i: (0, i, 0)),
            pl.BlockSpec((h, d_out), lambda i: (0, 0)),
            pl.BlockSpec((1, d_out), lambda i: (0, 0)),
        ],
        out_specs=pl.BlockSpec((blk, d_out), lambda i: (i, 0)),
        out_shape=jax.ShapeDtypeStruct((n, d_out), jnp.float32),
    )(partials, W_out, b_out.reshape(1, d_out))


def kernel(x, edge_attr, W_src, b_src, W_dst, b_dst, W_e1, b_e1, W_e2, b_e2,
           W_e3, b_e3, ln_g, ln_b, W_out, b_out, edge_index):
    n, _ = x.shape
    e, _ = edge_attr.shape
    h = W_src.shape[1]
    src = edge_index[0]
    dst = edge_index[1]
    W1u = W_e1[:h]
    W1v = W_e1[h:2 * h]
    W1e = W_e1[2 * h:]

    A, B = _node_tables(x, W_src, b_src, W_dst, b_dst, W1u, W1v)

    d_edge = edge_attr.shape[1]
    ea8 = edge_attr.reshape(e // 8, 8 * d_edge)
    W1e_big = jnp.zeros((8 * d_edge, 8 * h), jnp.float32)
    for k in range(8):
        W1e_big = W1e_big.at[k * d_edge:(k + 1) * d_edge,
                             k * h:(k + 1) * h].set(W1e)

    ep = e // _PARTS
    gather = _make_gather(n, ep, h)
    Gs = [gather(A, B, src[p * ep:(p + 1) * ep], dst[p * ep:(p + 1) * ep])
          for p in range(_PARTS)]
    hmsg = None
    for p in range(_PARTS):
        hmsg = _edge_mlp_part(hmsg, Gs[p], ea8, W1e_big, b_e1, W_e2, b_e2,
                              W_e3, b_e3, ln_g, ln_b, p, e)

    zeros = jnp.zeros((n, h), jnp.float32)
    partials = _make_scatter(n, e, h)(hmsg, dst, zeros)
    return _out_proj(partials, W_out, b_out)
